# baseline (speedup 1.0000x reference)
import functools

import jax
import jax.numpy as jnp
from jax import lax
from jax.experimental import pallas as pl
from jax.experimental.pallas import tpu as pltpu
from jax.experimental.pallas import tpu_sc as plsc

_B = 1024
_C = 100000
_M = 0.3
_S = 15.0

_BLOCK_C = 5000
_NBLOCKS = _C // _BLOCK_C

_NW = 32
_BPW = _B // _NW
_L = 16


def _tv_gather_body(x_hbm, tgt_hbm, out_hbm, tgt_v, idx_v, val_v, sem):
    wid = lax.axis_index("s") * 2 + lax.axis_index("c")
    base = wid * _BPW
    pltpu.sync_copy(tgt_hbm.at[pl.ds(base, _BPW)], tgt_v)
    lane = lax.iota(jnp.int32, _L)
    for c in range(_BPW // _L):
        r0 = c * _L
        t16 = tgt_v[pl.ds(r0, _L)]
        j16 = base + r0 + lane
        idx_v[pl.ds(r0, _L)] = (
            (t16 >> 3) * 8192
            + (t16 & 7) * 128
            + (j16 >> 7) * 1024
            + (j16 & 127)
        )
    pltpu.async_copy(x_hbm.at[idx_v], val_v, sem).wait()
    pltpu.sync_copy(val_v, out_hbm.at[pl.ds(base, _BPW)])


def _target_gather(x_flat, tgt):
    mesh = plsc.VectorSubcoreMesh(core_axis_name="c", subcore_axis_name="s")
    fn = functools.partial(
        pl.kernel,
        mesh=mesh,
        out_type=jax.ShapeDtypeStruct((_B,), jnp.float32),
        scratch_types=[
            pltpu.VMEM((_BPW,), jnp.int32),
            pltpu.VMEM((_BPW,), jnp.int32),
            pltpu.VMEM((_BPW,), jnp.float32),
            pltpu.SemaphoreType.DMA,
        ],
    )(_tv_gather_body)
    return fn(x_flat, tgt)


def _pass_kernel(x_ref, m_ref, s_ref):
    i = pl.program_id(0)

    @pl.when(i == 0)
    def _init():
        m_ref[...] = jnp.full((1, _B), -jnp.inf, jnp.float32)
        s_ref[...] = jnp.zeros((1, _B), jnp.float32)

    x = x_ref[...]

    bmax = jnp.max(x, axis=0, keepdims=True)
    m_old = m_ref[...]
    m_new = jnp.maximum(m_old, bmax)
    ex = jnp.exp(_S * x - _S * m_new)
    ones = jnp.ones((8, _BLOCK_C), jnp.float32)
    esum = jax.lax.dot_general(
        ones, ex, (((1,), (0,)), ((), ())), preferred_element_type=jnp.float32
    )[0:1]
    s_ref[...] = s_ref[...] * jnp.exp(_S * (m_old - m_new)) + esum
    m_ref[...] = m_new


def _combine_kernel(m_ref, s_ref, tv_ref, o_ref):
    m = m_ref[...]
    s = s_ref[...]
    tv = tv_ref[...]
    s_adj = s - jnp.exp(_S * (tv - m)) + jnp.exp(_S * (tv - _M - m))
    lse = _S * m + jnp.log(s_adj)
    loss = jnp.mean(lse - _S * (tv - _M))
    o_ref[...] = loss.reshape(1, 1)


@jax.jit
def _amsoftmax_loss(inputs, targets):
    xt = inputs.T
    tgt = targets.astype(jnp.int32)
    x_flat = xt.reshape(_C // 8, 8, _B // 128, 128).transpose(0, 2, 1, 3).reshape(-1)
    tv = _target_gather(x_flat, tgt)

    m, s = pl.pallas_call(
        _pass_kernel,
        grid=(_NBLOCKS,),
        in_specs=[pl.BlockSpec((_BLOCK_C, _B), lambda i: (i, 0))],
        out_specs=[
            pl.BlockSpec((1, _B), lambda i: (0, 0)),
            pl.BlockSpec((1, _B), lambda i: (0, 0)),
        ],
        out_shape=[
            jax.ShapeDtypeStruct((1, _B), jnp.float32),
            jax.ShapeDtypeStruct((1, _B), jnp.float32),
        ],
        compiler_params=pltpu.CompilerParams(
            dimension_semantics=("arbitrary",),
        ),
    )(xt)

    out = pl.pallas_call(
        _combine_kernel,
        out_shape=jax.ShapeDtypeStruct((1, 1), jnp.float32),
    )(m, s, tv.reshape(1, _B))
    return out[0, 0]


def kernel(inputs, targets):
    return _amsoftmax_loss(inputs, targets)


# speedup vs baseline: 1.0066x; 1.0066x over previous
import functools

import jax
import jax.numpy as jnp
from jax import lax
from jax.experimental import pallas as pl
from jax.experimental.pallas import tpu as pltpu
from jax.experimental.pallas import tpu_sc as plsc

_B = 1024
_C = 100000
_M = 0.3
_S = 15.0

_BLOCK_C = 4000
_NBLOCKS = _C // _BLOCK_C

_NW = 32
_BPW = _B // _NW
_L = 16


def _tv_gather_body(x_hbm, tgt_hbm, out_hbm, tgt_v, idx_v, val_v, sem):
    wid = lax.axis_index("s") * 2 + lax.axis_index("c")
    base = wid * _BPW
    pltpu.sync_copy(tgt_hbm.at[pl.ds(base, _BPW)], tgt_v)
    lane = lax.iota(jnp.int32, _L)
    for c in range(_BPW // _L):
        r0 = c * _L
        t16 = tgt_v[pl.ds(r0, _L)]
        j16 = base + r0 + lane
        idx_v[pl.ds(r0, _L)] = (
            (t16 >> 3) * 8192
            + (t16 & 7) * 128
            + (j16 >> 7) * 1024
            + (j16 & 127)
        )
    pltpu.async_copy(x_hbm.at[idx_v], val_v, sem).wait()
    pltpu.sync_copy(val_v, out_hbm.at[pl.ds(base, _BPW)])


def _target_gather(x_flat, tgt):
    mesh = plsc.VectorSubcoreMesh(core_axis_name="c", subcore_axis_name="s")
    fn = functools.partial(
        pl.kernel,
        mesh=mesh,
        out_type=jax.ShapeDtypeStruct((_B,), jnp.float32),
        scratch_types=[
            pltpu.VMEM((_BPW,), jnp.int32),
            pltpu.VMEM((_BPW,), jnp.int32),
            pltpu.VMEM((_BPW,), jnp.float32),
            pltpu.SemaphoreType.DMA,
        ],
    )(_tv_gather_body)
    return fn(x_flat, tgt)


def _pass_kernel(x_ref, m_ref, s_ref):
    i = pl.program_id(0)

    @pl.when(i == 0)
    def _init():
        m_ref[...] = jnp.full((1, _B), -jnp.inf, jnp.float32)
        s_ref[...] = jnp.zeros((1, _B), jnp.float32)

    x = x_ref[...]

    bmax = jnp.max(x, axis=0, keepdims=True)
    m_old = m_ref[...]
    m_new = jnp.maximum(m_old, bmax)
    ex = jnp.exp(_S * x - _S * m_new)
    ones = jnp.ones((8, _BLOCK_C), jnp.float32)
    esum = jax.lax.dot_general(
        ones, ex, (((1,), (0,)), ((), ())), preferred_element_type=jnp.float32
    )[0:1]
    s_ref[...] = s_ref[...] * jnp.exp(_S * (m_old - m_new)) + esum
    m_ref[...] = m_new


def _combine_kernel(m_ref, s_ref, tv_ref, o_ref):
    m = m_ref[...]
    s = s_ref[...]
    tv = tv_ref[...]
    s_adj = s - jnp.exp(_S * (tv - m)) + jnp.exp(_S * (tv - _M - m))
    lse = _S * m + jnp.log(s_adj)
    loss = jnp.mean(lse - _S * (tv - _M))
    o_ref[...] = loss.reshape(1, 1)


@jax.jit
def _amsoftmax_loss(inputs, targets):
    xt = inputs.T
    tgt = targets.astype(jnp.int32)
    x_flat = xt.reshape(_C // 8, 8, _B // 128, 128).transpose(0, 2, 1, 3).reshape(-1)
    tv = _target_gather(x_flat, tgt)

    m, s = pl.pallas_call(
        _pass_kernel,
        grid=(_NBLOCKS,),
        in_specs=[pl.BlockSpec((_BLOCK_C, _B), lambda i: (i, 0))],
        out_specs=[
            pl.BlockSpec((1, _B), lambda i: (0, 0)),
            pl.BlockSpec((1, _B), lambda i: (0, 0)),
        ],
        out_shape=[
            jax.ShapeDtypeStruct((1, _B), jnp.float32),
            jax.ShapeDtypeStruct((1, _B), jnp.float32),
        ],
        compiler_params=pltpu.CompilerParams(
            dimension_semantics=("arbitrary",),
        ),
    )(xt)

    out = pl.pallas_call(
        _combine_kernel,
        out_shape=jax.ShapeDtypeStruct((1, 1), jnp.float32),
    )(m, s, tv.reshape(1, _B))
    return out[0, 0]


def kernel(inputs, targets):
    return _amsoftmax_loss(inputs, targets)
